# baseline (speedup 1.0000x reference)
import jax
import jax.numpy as jnp
from jax.experimental import pallas as pl
from jax.experimental.pallas import tpu as pltpu
from jax.experimental.pallas import tpu_sc as plsc

_WINDOW = 256
_TBLK = 4096


def _transpose_table(weight):
    wt = weight.T
    d, v = wt.shape
    grid = (v + _TBLK - 1) // _TBLK

    def body(in_ref, out_ref):
        out_ref[...] = in_ref[...].T

    return pl.pallas_call(
        body,
        grid=(grid,),
        in_specs=[pl.BlockSpec((d, _TBLK), lambda i: (0, i))],
        out_specs=pl.BlockSpec((_TBLK, d), lambda i: (i, 0)),
        out_shape=jax.ShapeDtypeStruct((v, d), wt.dtype),
        compiler_params=pltpu.CompilerParams(dimension_semantics=("parallel",)),
    )(wt)


def _gather_rows(weight, idx2d, n, dim):
    mesh = plsc.VectorSubcoreMesh(core_axis_name="core", subcore_axis_name="subcore")

    @pl.kernel(
        out_type=jax.ShapeDtypeStruct((n, dim), weight.dtype),
        mesh=mesh,
        compiler_params=pltpu.CompilerParams(use_tc_tiling_on_sc=False),
    )
    def gather_kernel(w_hbm, i_hbm, o_hbm):
        def body(i_vmem, o_vmem):
            pltpu.sync_copy(w_hbm.at[i_vmem.at[0]], o_vmem)

        pltpu.emit_pipeline(
            body,
            grid=(n // _WINDOW,),
            in_specs=[pl.BlockSpec((1, _WINDOW), index_map=lambda i: (0, i))],
            out_specs=[pl.BlockSpec((_WINDOW, dim), index_map=lambda i: (i, 0))],
            core_axis_name=("core", "subcore"),
            dimension_semantics=(pltpu.PARALLEL,),
        )(i_hbm, o_hbm)

    return gather_kernel(weight, idx2d)


def _transpose_out(g, s, b, d):
    g3 = g.reshape(s, b, d)

    def body(in_ref, out_ref):
        out_ref[0] = in_ref[0].T

    return pl.pallas_call(
        body,
        grid=(s,),
        in_specs=[pl.BlockSpec((1, b, d), lambda i: (i, 0, 0))],
        out_specs=pl.BlockSpec((1, d, b), lambda i: (i, 0, 0)),
        out_shape=jax.ShapeDtypeStruct((s, d, b), g.dtype),
        compiler_params=pltpu.CompilerParams(dimension_semantics=("parallel",)),
    )(g3)


def kernel(x, weight):
    b, s = x.shape
    n = b * s
    d = weight.shape[1]
    idx2d = x.T.reshape(1, n).astype(jnp.int32)
    w_rm = _transpose_table(weight)
    g = _gather_rows(w_rm, idx2d, n, d)
    out_p = _transpose_out(g, s, b, d)
    return jnp.transpose(out_p, (2, 0, 1))


# speedup vs baseline: 1.0977x; 1.0977x over previous
import jax
import jax.numpy as jnp
from jax.experimental import pallas as pl
from jax.experimental.pallas import tpu as pltpu
from jax.experimental.pallas import tpu_sc as plsc

_WINDOW = 256
_TBLK = 4096


def _transpose_table(weight):
    wt = weight.T
    d, v = wt.shape
    grid = (v + _TBLK - 1) // _TBLK

    def body(in_ref, out_ref):
        out_ref[...] = in_ref[...].T

    return pl.pallas_call(
        body,
        grid=(grid,),
        in_specs=[pl.BlockSpec((d, _TBLK), lambda i: (0, i))],
        out_specs=pl.BlockSpec((_TBLK, d), lambda i: (i, 0)),
        out_shape=jax.ShapeDtypeStruct((v, d), wt.dtype),
        compiler_params=pltpu.CompilerParams(dimension_semantics=("parallel",)),
    )(wt)


def _gather_rows(weight, idx2d, n, dim):
    mesh = plsc.VectorSubcoreMesh(core_axis_name="core", subcore_axis_name="subcore")

    @pl.kernel(
        out_type=jax.ShapeDtypeStruct((n, dim), weight.dtype),
        mesh=mesh,
        compiler_params=pltpu.CompilerParams(use_tc_tiling_on_sc=False),
    )
    def gather_kernel(w_hbm, i_hbm, o_hbm):
        def body(i_vmem, o_vmem):
            pltpu.sync_copy(w_hbm.at[i_vmem.at[0]], o_vmem)

        pltpu.emit_pipeline(
            body,
            grid=(n // _WINDOW,),
            in_specs=[pl.BlockSpec((1, _WINDOW), index_map=lambda i: (0, i))],
            out_specs=[pl.BlockSpec((_WINDOW, dim), index_map=lambda i: (i, 0))],
            core_axis_name=("core", "subcore"),
            dimension_semantics=(pltpu.PARALLEL,),
        )(i_hbm, o_hbm)

    return gather_kernel(weight, idx2d)


def _transpose_out(g, s, b, d):
    g3 = g.reshape(s, b, d)

    def body(in_ref, out_ref):
        out_ref[0] = in_ref[0].T

    return pl.pallas_call(
        body,
        grid=(s,),
        in_specs=[pl.BlockSpec((1, b, d), lambda i: (i, 0, 0))],
        out_specs=pl.BlockSpec((1, d, b), lambda i: (i, 0, 0)),
        out_shape=jax.ShapeDtypeStruct((s, d, b), g.dtype),
        compiler_params=pltpu.CompilerParams(dimension_semantics=("parallel",)),
    )(g3)


def kernel(x, weight):
    b, s = x.shape
    n = b * s
    d = weight.shape[1]
    idx2d = x.T.reshape(1, n).astype(jnp.int32)
    w_rm = _transpose_table(weight)
    g = _gather_rows(w_rm, idx2d, n, d)
    return jnp.transpose(g.reshape(s, b, d), (1, 0, 2))


# speedup vs baseline: 1.1224x; 1.0225x over previous
import jax
import jax.numpy as jnp
from jax.experimental import pallas as pl
from jax.experimental.pallas import tpu as pltpu
from jax.experimental.pallas import tpu_sc as plsc

_WINDOW = 256
_TBLK = 16384
_OSLAB = 4


def _transpose_table(weight):
    wt = weight.T
    d, v = wt.shape
    grid = (v + _TBLK - 1) // _TBLK

    def body(in_ref, out_ref):
        out_ref[...] = in_ref[...].T

    return pl.pallas_call(
        body,
        grid=(grid,),
        in_specs=[pl.BlockSpec((d, _TBLK), lambda i: (0, i))],
        out_specs=pl.BlockSpec((_TBLK, d), lambda i: (i, 0)),
        out_shape=jax.ShapeDtypeStruct((v, d), wt.dtype),
        compiler_params=pltpu.CompilerParams(dimension_semantics=("parallel",)),
    )(wt)


def _gather_rows(weight, idx2d, n, dim):
    mesh = plsc.VectorSubcoreMesh(core_axis_name="core", subcore_axis_name="subcore")

    @pl.kernel(
        out_type=jax.ShapeDtypeStruct((n, dim), weight.dtype),
        mesh=mesh,
        compiler_params=pltpu.CompilerParams(use_tc_tiling_on_sc=False),
    )
    def gather_kernel(w_hbm, i_hbm, o_hbm):
        def body(i_vmem, o_vmem):
            pltpu.sync_copy(w_hbm.at[i_vmem.at[0]], o_vmem)

        pltpu.emit_pipeline(
            body,
            grid=(n // _WINDOW,),
            in_specs=[pl.BlockSpec((1, _WINDOW), index_map=lambda i: (0, i))],
            out_specs=[pl.BlockSpec((_WINDOW, dim), index_map=lambda i: (i, 0))],
            core_axis_name=("core", "subcore"),
            dimension_semantics=(pltpu.PARALLEL,),
        )(i_hbm, o_hbm)

    return gather_kernel(weight, idx2d)


def _transpose_out(g, s, b, d):
    g3 = g.reshape(s, b, d)

    def body(in_ref, out_ref):
        for j in range(_OSLAB):
            out_ref[j] = in_ref[j].T

    return pl.pallas_call(
        body,
        grid=(s // _OSLAB,),
        in_specs=[pl.BlockSpec((_OSLAB, b, d), lambda i: (i, 0, 0))],
        out_specs=pl.BlockSpec((_OSLAB, d, b), lambda i: (i, 0, 0)),
        out_shape=jax.ShapeDtypeStruct((s, d, b), g.dtype),
        compiler_params=pltpu.CompilerParams(dimension_semantics=("parallel",)),
    )(g3)


def kernel(x, weight):
    b, s = x.shape
    n = b * s
    d = weight.shape[1]
    idx2d = x.T.reshape(1, n).astype(jnp.int32)
    w_rm = _transpose_table(weight)
    g = _gather_rows(w_rm, idx2d, n, d)
    out_p = _transpose_out(g, s, b, d)
    return jnp.transpose(out_p, (2, 0, 1))
